# baseline (device time: 77272 ns/iter reference)
import os

import jax
import jax.numpy as jnp
from jax import lax
from jax.experimental import pallas as pl
from jax.experimental.pallas import tpu as pltpu

_ENABLE_P1 = os.environ.get("AG_DISABLE_P1") != "1"
_ENABLE_INJ = os.environ.get("AG_DISABLE_INJ") != "1"
_ENABLE_OWN = os.environ.get("AG_DISABLE_OWN") != "1"

N_DEV = 32
N_H = 8
N_J = 7
N_SEG = 4


def _mesh_order():
    coords = []
    for z in range(4):
        for yi, y in enumerate(range(4)):
            row = [(x, y, z) for x in range(2)]
            if yi % 2:
                row.reverse()
            coords.extend(row)
    return coords


def _hamiltonian_ring():
    path = []
    for zi, z in enumerate(range(4)):
        ys = list(range(4))
        if zi % 2:
            ys.reverse()
        path.extend((y, z) for y in ys)
    ring = [(1, y, z) for (y, z) in path]
    ring += [(0, y, z) for (y, z) in reversed(path)]
    return ring


_MESH_IDX = {c: i for i, c in enumerate(_mesh_order())}
_PERM = [_MESH_IDX[c] for c in _hamiltonian_ring()]
_INV = [0] * N_DEV
for _r, _m in enumerate(_PERM):
    _INV[_m] = _r


def kernel(x):
    m_per, n = x.shape

    perm = jnp.array(_PERM, dtype=jnp.int32)
    inv = jnp.array(_INV, dtype=jnp.int32)

    my = lax.axis_index("i")
    r = inv[my]
    nbrs = jnp.stack([
        perm[(r - 1) % N_DEV],
        perm[(r + 1) % N_DEV],
        perm[(r + 16) % N_DEV],
    ]).astype(jnp.int32)

    hh = jnp.arange(N_H, dtype=jnp.int32)
    hj = jnp.arange(N_J, dtype=jnp.int32)
    s_r_own = perm[(r - hh) % N_DEV]
    s_r_inj = perm[(r + 16 - hj) % N_DEV]
    s_l_own = perm[(r + hh) % N_DEV]
    s_l_inj = perm[(r + 16 + hj) % N_DEV]
    r_r_own = perm[(r - 1 - hh) % N_DEV]
    r_r_inj = perm[(r + 15 - hj) % N_DEV]
    r_l_own = perm[(r + 1 + hh) % N_DEV]
    r_l_inj = perm[(r + 17 + hj) % N_DEV]

    def body(x_ref, nbrs_ref,
             s_r_own_ref, s_r_inj_ref, s_l_own_ref, s_l_inj_ref,
             r_r_own_ref, r_r_inj_ref, r_l_own_ref, r_l_inj_ref,
             out_ref,
             p1_send, p1_recv,
             snd_r_own, rcv_r_own, snd_r_inj, rcv_r_inj,
             snd_l_own, rcv_l_own, snd_l_inj, rcv_l_inj):
        me = lax.axis_index("i")
        lft = nbrs_ref[0]
        rgt = nbrs_ref[1]
        anti = nbrs_ref[2]

        barrier_sem = pltpu.get_barrier_semaphore()
        for nbr in (lft, rgt, anti):
            pl.semaphore_signal(
                barrier_sem, inc=1,
                device_id=(nbr,), device_id_type=pl.DeviceIdType.MESH,
            )
        pl.semaphore_wait(barrier_sem, 3)

        seg_rows = m_per // N_SEG

        out_ref[pl.ds(me * m_per, m_per), :] = x_ref[...]

        sends = []

        for s in range(N_SEG) if _ENABLE_P1 else ():
            p1 = pltpu.make_async_remote_copy(
                src_ref=x_ref.at[pl.ds(s * seg_rows, seg_rows), :],
                dst_ref=out_ref.at[pl.ds(me * m_per + s * seg_rows,
                                         seg_rows), :],
                send_sem=p1_send.at[s],
                recv_sem=p1_recv.at[s],
                device_id=(anti,),
                device_id_type=pl.DeviceIdType.MESH,
            )
            p1.start()
            sends.append(p1)

        def send(tab_ref, snd, rcv, h, s, target):
            off = tab_ref[h] * m_per + s * seg_rows
            rdma = pltpu.make_async_remote_copy(
                src_ref=out_ref.at[pl.ds(off, seg_rows), :],
                dst_ref=out_ref.at[pl.ds(off, seg_rows), :],
                send_sem=snd.at[h, s],
                recv_sem=rcv.at[h, s],
                device_id=(target,),
                device_id_type=pl.DeviceIdType.MESH,
            )
            rdma.start()
            sends.append(rdma)

        def wait_recv(tab_ref, snd, rcv, h, s, src_dev):
            off = tab_ref[h] * m_per + s * seg_rows
            rdma = pltpu.make_async_remote_copy(
                src_ref=out_ref.at[pl.ds(off, seg_rows), :],
                dst_ref=out_ref.at[pl.ds(off, seg_rows), :],
                send_sem=snd.at[h, s],
                recv_sem=rcv.at[h, s],
                device_id=(src_dev,),
                device_id_type=pl.DeviceIdType.MESH,
            )
            rdma.wait_recv()

        for s in range(N_SEG) if _ENABLE_P1 else ():
            p1w = pltpu.make_async_remote_copy(
                src_ref=x_ref.at[pl.ds(s * seg_rows, seg_rows), :],
                dst_ref=out_ref.at[pl.ds(
                    s_r_inj_ref[0] * m_per + s * seg_rows, seg_rows), :],
                send_sem=p1_send.at[s],
                recv_sem=p1_recv.at[s],
                device_id=(anti,),
                device_id_type=pl.DeviceIdType.MESH,
            )
            p1w.wait_recv()
        if _ENABLE_P1:
            for p1 in sends:
                p1.wait_send()
            sends = []
            for nbr in (lft, rgt, anti):
                pl.semaphore_signal(
                    barrier_sem, inc=1,
                    device_id=(nbr,), device_id_type=pl.DeviceIdType.MESH,
                )
            pl.semaphore_wait(barrier_sem, 3)

        for s in range(N_SEG) if _ENABLE_OWN else ():
            send(s_r_own_ref, snd_r_own, rcv_r_own, 0, s, rgt)
            send(s_l_own_ref, snd_l_own, rcv_l_own, 0, s, lft)
        for s in range(N_SEG) if _ENABLE_INJ else ():
            send(s_r_inj_ref, snd_r_inj, rcv_r_inj, 0, s, rgt)
            send(s_l_inj_ref, snd_l_inj, rcv_l_inj, 0, s, lft)

        for h in range(1, N_H) if _ENABLE_OWN else ():
            for s in range(N_SEG):
                wait_recv(r_r_own_ref, snd_r_own, rcv_r_own, h - 1, s, lft)
                send(s_r_own_ref, snd_r_own, rcv_r_own, h, s, rgt)
                wait_recv(r_l_own_ref, snd_l_own, rcv_l_own, h - 1, s, rgt)
                send(s_l_own_ref, snd_l_own, rcv_l_own, h, s, lft)
                if h < N_J and _ENABLE_INJ:
                    wait_recv(r_r_inj_ref, snd_r_inj, rcv_r_inj, h - 1, s, lft)
                    send(s_r_inj_ref, snd_r_inj, rcv_r_inj, h, s, rgt)
                    wait_recv(r_l_inj_ref, snd_l_inj, rcv_l_inj, h - 1, s, rgt)
                    send(s_l_inj_ref, snd_l_inj, rcv_l_inj, h, s, lft)

        for s in range(N_SEG) if _ENABLE_OWN else ():
            wait_recv(r_r_own_ref, snd_r_own, rcv_r_own, N_H - 1, s, lft)
            wait_recv(r_l_own_ref, snd_l_own, rcv_l_own, N_H - 1, s, rgt)
            if _ENABLE_INJ:
                wait_recv(r_r_inj_ref, snd_r_inj, rcv_r_inj, N_J - 1, s, lft)
                wait_recv(r_l_inj_ref, snd_l_inj, rcv_l_inj, N_J - 1, s, rgt)

        for rdma in sends:
            rdma.wait_send()

    smem = pl.BlockSpec(memory_space=pltpu.SMEM)
    return pl.pallas_call(
        body,
        out_shape=jax.ShapeDtypeStruct((N_DEV * m_per, n), x.dtype),
        in_specs=[pl.BlockSpec(memory_space=pltpu.VMEM)] + [smem] * 9,
        out_specs=pl.BlockSpec(memory_space=pltpu.VMEM),
        scratch_shapes=[
            pltpu.SemaphoreType.DMA((N_SEG,)),
            pltpu.SemaphoreType.DMA((N_SEG,)),
            pltpu.SemaphoreType.DMA((N_H, N_SEG)),
            pltpu.SemaphoreType.DMA((N_H, N_SEG)),
            pltpu.SemaphoreType.DMA((N_J, N_SEG)),
            pltpu.SemaphoreType.DMA((N_J, N_SEG)),
            pltpu.SemaphoreType.DMA((N_H, N_SEG)),
            pltpu.SemaphoreType.DMA((N_H, N_SEG)),
            pltpu.SemaphoreType.DMA((N_J, N_SEG)),
            pltpu.SemaphoreType.DMA((N_J, N_SEG)),
        ],
        compiler_params=pltpu.CompilerParams(collective_id=0),
    )(x, nbrs, s_r_own, s_r_inj, s_l_own, s_l_inj,
      r_r_own, r_r_inj, r_l_own, r_l_inj)


# device time: 76277 ns/iter; 1.0130x vs baseline; 1.0130x over previous
import os

import jax
import jax.numpy as jnp
from jax import lax
from jax.experimental import pallas as pl
from jax.experimental.pallas import tpu as pltpu

_ENABLE_P1 = os.environ.get("AG_DISABLE_P1") != "1"
_ENABLE_INJ = os.environ.get("AG_DISABLE_INJ") != "1"
_ENABLE_OWN = os.environ.get("AG_DISABLE_OWN") != "1"

N_DEV = 32
N_H = 8
N_J = 7
N_SEG = 2


def _mesh_order():
    coords = []
    for z in range(4):
        for yi, y in enumerate(range(4)):
            row = [(x, y, z) for x in range(2)]
            if yi % 2:
                row.reverse()
            coords.extend(row)
    return coords


def _hamiltonian_ring():
    path = []
    for zi, z in enumerate(range(4)):
        ys = list(range(4))
        if zi % 2:
            ys.reverse()
        path.extend((y, z) for y in ys)
    ring = [(1, y, z) for (y, z) in path]
    ring += [(0, y, z) for (y, z) in reversed(path)]
    return ring


_MESH_IDX = {c: i for i, c in enumerate(_mesh_order())}
_PERM = [_MESH_IDX[c] for c in _hamiltonian_ring()]
_INV = [0] * N_DEV
for _r, _m in enumerate(_PERM):
    _INV[_m] = _r


def kernel(x):
    m_per, n = x.shape

    perm = jnp.array(_PERM, dtype=jnp.int32)
    inv = jnp.array(_INV, dtype=jnp.int32)

    my = lax.axis_index("i")
    r = inv[my]
    nbrs = jnp.stack([
        perm[(r - 1) % N_DEV],
        perm[(r + 1) % N_DEV],
        perm[(r + 16) % N_DEV],
    ]).astype(jnp.int32)

    hh = jnp.arange(N_H, dtype=jnp.int32)
    hj = jnp.arange(N_J, dtype=jnp.int32)
    s_r_own = perm[(r - hh) % N_DEV]
    s_r_inj = perm[(r + 16 - hj) % N_DEV]
    s_l_own = perm[(r + hh) % N_DEV]
    s_l_inj = perm[(r + 16 + hj) % N_DEV]
    r_r_own = perm[(r - 1 - hh) % N_DEV]
    r_r_inj = perm[(r + 15 - hj) % N_DEV]
    r_l_own = perm[(r + 1 + hh) % N_DEV]
    r_l_inj = perm[(r + 17 + hj) % N_DEV]

    def body(x_ref, nbrs_ref,
             s_r_own_ref, s_r_inj_ref, s_l_own_ref, s_l_inj_ref,
             r_r_own_ref, r_r_inj_ref, r_l_own_ref, r_l_inj_ref,
             out_ref,
             p1_send, p1_recv,
             snd_r_own, rcv_r_own, snd_r_inj, rcv_r_inj,
             snd_l_own, rcv_l_own, snd_l_inj, rcv_l_inj):
        me = lax.axis_index("i")
        lft = nbrs_ref[0]
        rgt = nbrs_ref[1]
        anti = nbrs_ref[2]

        barrier_sem = pltpu.get_barrier_semaphore()
        for nbr in (lft, rgt, anti):
            pl.semaphore_signal(
                barrier_sem, inc=1,
                device_id=(nbr,), device_id_type=pl.DeviceIdType.MESH,
            )
        pl.semaphore_wait(barrier_sem, 3)

        seg_rows = m_per // N_SEG

        out_ref[pl.ds(me * m_per, m_per), :] = x_ref[...]

        sends = []

        for s in range(N_SEG) if _ENABLE_P1 else ():
            p1 = pltpu.make_async_remote_copy(
                src_ref=x_ref.at[pl.ds(s * seg_rows, seg_rows), :],
                dst_ref=out_ref.at[pl.ds(me * m_per + s * seg_rows,
                                         seg_rows), :],
                send_sem=p1_send.at[s],
                recv_sem=p1_recv.at[s],
                device_id=(anti,),
                device_id_type=pl.DeviceIdType.MESH,
            )
            p1.start()
            sends.append(p1)

        def send(tab_ref, snd, rcv, h, s, target):
            off = tab_ref[h] * m_per + s * seg_rows
            rdma = pltpu.make_async_remote_copy(
                src_ref=out_ref.at[pl.ds(off, seg_rows), :],
                dst_ref=out_ref.at[pl.ds(off, seg_rows), :],
                send_sem=snd.at[h, s],
                recv_sem=rcv.at[h, s],
                device_id=(target,),
                device_id_type=pl.DeviceIdType.MESH,
            )
            rdma.start()
            sends.append(rdma)

        def wait_recv(tab_ref, snd, rcv, h, s, src_dev):
            off = tab_ref[h] * m_per + s * seg_rows
            rdma = pltpu.make_async_remote_copy(
                src_ref=out_ref.at[pl.ds(off, seg_rows), :],
                dst_ref=out_ref.at[pl.ds(off, seg_rows), :],
                send_sem=snd.at[h, s],
                recv_sem=rcv.at[h, s],
                device_id=(src_dev,),
                device_id_type=pl.DeviceIdType.MESH,
            )
            rdma.wait_recv()

        for s in range(N_SEG) if _ENABLE_P1 else ():
            p1w = pltpu.make_async_remote_copy(
                src_ref=x_ref.at[pl.ds(s * seg_rows, seg_rows), :],
                dst_ref=out_ref.at[pl.ds(
                    s_r_inj_ref[0] * m_per + s * seg_rows, seg_rows), :],
                send_sem=p1_send.at[s],
                recv_sem=p1_recv.at[s],
                device_id=(anti,),
                device_id_type=pl.DeviceIdType.MESH,
            )
            p1w.wait_recv()
        if _ENABLE_P1:
            for nbr in (lft, rgt, anti):
                pl.semaphore_signal(
                    barrier_sem, inc=1,
                    device_id=(nbr,), device_id_type=pl.DeviceIdType.MESH,
                )
            pl.semaphore_wait(barrier_sem, 3)

        for s in range(N_SEG) if _ENABLE_OWN else ():
            send(s_r_own_ref, snd_r_own, rcv_r_own, 0, s, rgt)
            send(s_l_own_ref, snd_l_own, rcv_l_own, 0, s, lft)
        for s in range(N_SEG) if _ENABLE_INJ else ():
            send(s_r_inj_ref, snd_r_inj, rcv_r_inj, 0, s, rgt)
            send(s_l_inj_ref, snd_l_inj, rcv_l_inj, 0, s, lft)

        for h in range(1, N_H) if _ENABLE_OWN else ():
            for s in range(N_SEG):
                wait_recv(r_r_own_ref, snd_r_own, rcv_r_own, h - 1, s, lft)
                send(s_r_own_ref, snd_r_own, rcv_r_own, h, s, rgt)
                wait_recv(r_l_own_ref, snd_l_own, rcv_l_own, h - 1, s, rgt)
                send(s_l_own_ref, snd_l_own, rcv_l_own, h, s, lft)
                if h < N_J and _ENABLE_INJ:
                    wait_recv(r_r_inj_ref, snd_r_inj, rcv_r_inj, h - 1, s, lft)
                    send(s_r_inj_ref, snd_r_inj, rcv_r_inj, h, s, rgt)
                    wait_recv(r_l_inj_ref, snd_l_inj, rcv_l_inj, h - 1, s, rgt)
                    send(s_l_inj_ref, snd_l_inj, rcv_l_inj, h, s, lft)

        for s in range(N_SEG) if _ENABLE_OWN else ():
            wait_recv(r_r_own_ref, snd_r_own, rcv_r_own, N_H - 1, s, lft)
            wait_recv(r_l_own_ref, snd_l_own, rcv_l_own, N_H - 1, s, rgt)
            if _ENABLE_INJ:
                wait_recv(r_r_inj_ref, snd_r_inj, rcv_r_inj, N_J - 1, s, lft)
                wait_recv(r_l_inj_ref, snd_l_inj, rcv_l_inj, N_J - 1, s, rgt)

        for rdma in sends:
            rdma.wait_send()

    smem = pl.BlockSpec(memory_space=pltpu.SMEM)
    return pl.pallas_call(
        body,
        out_shape=jax.ShapeDtypeStruct((N_DEV * m_per, n), x.dtype),
        in_specs=[pl.BlockSpec(memory_space=pltpu.VMEM)] + [smem] * 9,
        out_specs=pl.BlockSpec(memory_space=pltpu.VMEM),
        scratch_shapes=[
            pltpu.SemaphoreType.DMA((N_SEG,)),
            pltpu.SemaphoreType.DMA((N_SEG,)),
            pltpu.SemaphoreType.DMA((N_H, N_SEG)),
            pltpu.SemaphoreType.DMA((N_H, N_SEG)),
            pltpu.SemaphoreType.DMA((N_J, N_SEG)),
            pltpu.SemaphoreType.DMA((N_J, N_SEG)),
            pltpu.SemaphoreType.DMA((N_H, N_SEG)),
            pltpu.SemaphoreType.DMA((N_H, N_SEG)),
            pltpu.SemaphoreType.DMA((N_J, N_SEG)),
            pltpu.SemaphoreType.DMA((N_J, N_SEG)),
        ],
        compiler_params=pltpu.CompilerParams(collective_id=0),
    )(x, nbrs, s_r_own, s_r_inj, s_l_own, s_l_inj,
      r_r_own, r_r_inj, r_l_own, r_l_inj)


# device time: 65474 ns/iter; 1.1802x vs baseline; 1.1650x over previous
import jax
import jax.numpy as jnp
from jax import lax
from jax.experimental import pallas as pl
from jax.experimental.pallas import tpu as pltpu

N_DEV = 32
N_R = N_DEV // 2
N_L = N_DEV - 1 - N_R
N_SEG = 4


def _mesh_order():
    coords = []
    for z in range(4):
        for yi, y in enumerate(range(4)):
            row = [(x, y, z) for x in range(2)]
            if yi % 2:
                row.reverse()
            coords.extend(row)
    return coords


def _hamiltonian_ring():
    path = []
    for zi, z in enumerate(range(4)):
        ys = list(range(4))
        if zi % 2:
            ys.reverse()
        path.extend((y, z) for y in ys)
    ring = [(1, y, z) for (y, z) in path]
    ring += [(0, y, z) for (y, z) in reversed(path)]
    return ring


_MESH_IDX = {c: i for i, c in enumerate(_mesh_order())}
_PERM = [_MESH_IDX[c] for c in _hamiltonian_ring()]
_INV = [0] * N_DEV
for _r, _m in enumerate(_PERM):
    _INV[_m] = _r


def kernel(x):
    m_per, n = x.shape

    perm = jnp.array(_PERM, dtype=jnp.int32)
    inv = jnp.array(_INV, dtype=jnp.int32)

    my = lax.axis_index("i")
    r = inv[my]
    right = perm[(r + 1) % N_DEV]
    left = perm[(r - 1) % N_DEV]
    nbrs = jnp.stack([left, right]).astype(jnp.int32)

    hr = jnp.arange(N_R, dtype=jnp.int32)
    hl = jnp.arange(N_L, dtype=jnp.int32)
    orig_r = perm[(r - hr) % N_DEV]
    orig_l = perm[(r + hl) % N_DEV]
    rcv_r = perm[(r - 1 - hr) % N_DEV]
    rcv_l = perm[(r + 1 + hl) % N_DEV]

    def body(x_ref, nbrs_ref, orig_r_ref, orig_l_ref, rcv_r_ref, rcv_l_ref,
             out_ref, send_r, recv_r, send_l, recv_l):
        me = lax.axis_index("i")
        lft = nbrs_ref[0]
        rgt = nbrs_ref[1]

        barrier_sem = pltpu.get_barrier_semaphore()
        for nbr in (lft, rgt):
            pl.semaphore_signal(
                barrier_sem, inc=1,
                device_id=(nbr,), device_id_type=pl.DeviceIdType.MESH,
            )
        pl.semaphore_wait(barrier_sem, 2)

        out_ref[pl.ds(me * m_per, m_per), :] = x_ref[...]

        seg_rows = m_per // N_SEG

        def send(h, s, to_right):
            origin = orig_r_ref[h] if to_right else orig_l_ref[h]
            off = origin * m_per + s * seg_rows
            rdma = pltpu.make_async_remote_copy(
                src_ref=out_ref.at[pl.ds(off, seg_rows), :],
                dst_ref=out_ref.at[pl.ds(off, seg_rows), :],
                send_sem=(send_r if to_right else send_l).at[h, s],
                recv_sem=(recv_r if to_right else recv_l).at[h, s],
                device_id=((rgt if to_right else lft),),
                device_id_type=pl.DeviceIdType.MESH,
            )
            rdma.start()
            return rdma

        def wait_recv(h, s, from_left):
            origin = rcv_r_ref[h] if from_left else rcv_l_ref[h]
            off = origin * m_per + s * seg_rows
            rdma = pltpu.make_async_remote_copy(
                src_ref=out_ref.at[pl.ds(off, seg_rows), :],
                dst_ref=out_ref.at[pl.ds(off, seg_rows), :],
                send_sem=(send_r if from_left else send_l).at[h, s],
                recv_sem=(recv_r if from_left else recv_l).at[h, s],
                device_id=((lft if from_left else rgt),),
                device_id_type=pl.DeviceIdType.MESH,
            )
            rdma.wait_recv()

        sends = []
        for s in range(N_SEG):
            sends.append(send(0, s, True))
            sends.append(send(0, s, False))
        for h in range(1, N_R):
            for s in range(N_SEG):
                wait_recv(h - 1, s, True)
                sends.append(send(h, s, True))
                if h < N_L:
                    wait_recv(h - 1, s, False)
                    sends.append(send(h, s, False))
        for s in range(N_SEG):
            wait_recv(N_R - 1, s, True)
        for s in range(N_SEG):
            wait_recv(N_L - 1, s, False)

        for rdma in sends:
            rdma.wait_send()

    smem = pl.BlockSpec(memory_space=pltpu.SMEM)
    return pl.pallas_call(
        body,
        out_shape=jax.ShapeDtypeStruct((N_DEV * m_per, n), x.dtype),
        in_specs=[
            pl.BlockSpec(memory_space=pltpu.VMEM),
            smem, smem, smem, smem, smem,
        ],
        out_specs=pl.BlockSpec(memory_space=pltpu.VMEM),
        scratch_shapes=[
            pltpu.SemaphoreType.DMA((N_R, N_SEG)),
            pltpu.SemaphoreType.DMA((N_R, N_SEG)),
            pltpu.SemaphoreType.DMA((N_L, N_SEG)),
            pltpu.SemaphoreType.DMA((N_L, N_SEG)),
        ],
        compiler_params=pltpu.CompilerParams(collective_id=0),
    )(x, nbrs, orig_r, orig_l, rcv_r, rcv_l)
